# Initial kernel scaffold; baseline (speedup 1.0000x reference)
import jax
import jax.numpy as jnp
from jax.experimental import pallas as pl


def kernel(x, g_ret, g_sto, Wq, Wk, Wv, W_lr, b_lr, W_mom, b_mom, W_dec, b_dec, W_gate, b_gate, W_comb, b_comb, mw1, mw2, mg, mb):
    raise NotImplementedError("write your pallas kernel here")


# speedup vs baseline: 8.1126x; 8.1126x over previous
import jax
import jax.numpy as jnp
from jax.experimental import pallas as pl
from jax.experimental.pallas import tpu as pltpu

DIM = 512
HEADS = 4
DH = 128
HID = 512
CHUNK = 64
MAX_LR = 0.01
EPS = 1e-6
_INV_SQRT2 = 0.7071067811865476
_INV_SQRT_2PI = 0.3989422804014327


def _prep_body(x_ref, gs_ref, gr_ref, wq_ref, wk_ref, wv_ref, wsm_ref, bsm_ref,
               k_ref, v_ref, q_ref, lrg_ref, md_ref):
    xc = x_ref[0]
    rs = jax.lax.rsqrt(jnp.mean(xc * xc, axis=-1, keepdims=True) + EPS)
    xs = xc * rs * gs_ref[...]
    xr = xc * rs * gr_ref[...]
    kall = jnp.dot(xs, wk_ref[...], preferred_element_type=jnp.float32)
    vall = jnp.dot(xs, wv_ref[...], preferred_element_type=jnp.float32)
    qall = jnp.dot(xr, wq_ref[...], preferred_element_type=jnp.float32)
    zs = jnp.dot(xs, wsm_ref[...], preferred_element_type=jnp.float32) + bsm_ref[...]
    zr = jnp.dot(xr, wsm_ref[...], preferred_element_type=jnp.float32) + bsm_ref[...]
    pooled = jnp.mean(xs, axis=0, keepdims=True)
    zp = jnp.dot(pooled, wsm_ref[...], preferred_element_type=jnp.float32) + bsm_ref[...]
    for h in range(HEADS):
        sl = slice(h * DH, (h + 1) * DH)
        kh = kall[:, sl]
        k_ref[0, h, 0] = kh * jax.lax.rsqrt(jnp.sum(kh * kh, -1, keepdims=True) + 1e-12)
        qh = qall[:, sl]
        q_ref[0, h, 0] = qh * jax.lax.rsqrt(jnp.sum(qh * qh, -1, keepdims=True) + 1e-12)
        v_ref[0, h, 0] = vall[:, sl]
    lr = jax.nn.sigmoid(zs[:, 0:HEADS]) * (2.0 * MAX_LR / DH)
    gate = jax.nn.sigmoid(zr[:, HEADS:2 * HEADS])
    lrg_ref[0, 0] = jnp.concatenate([lr, gate], axis=-1)
    md_ref[0, 0] = jax.nn.sigmoid(zp[:, 2 * HEADS:4 * HEADS])


def _main_body(k_ref, v_ref, q_ref, lr_ref, g_ref, md_ref, mw1_ref, mw2_ref,
               mgb_ref, o_ref, w1c, m1, w2c, m2, gbc):
    p = pl.program_id(0)
    nc = k_ref.shape[2]
    w1c[...] = mw1_ref[0]
    m1[...] = jnp.zeros_like(m1)
    w2c[...] = mw2_ref[0]
    m2[...] = jnp.zeros_like(m2)
    gbc[0:2, :] = mgb_ref[0]
    gbc[2:4, :] = jnp.zeros((2, DH), jnp.float32)
    mw1 = mw1_ref[0]
    mw2 = mw2_ref[0]
    mg_r = mgb_ref[0, 0:1, :]
    mb_r = mgb_ref[0, 1:2, :]

    def step(i, carry):
        q = q_ref[0, 0, i]
        u = jnp.dot(q, w1c[...], preferred_element_type=jnp.float32)
        cu = 0.5 * (1.0 + jax.lax.erf(u * _INV_SQRT2))
        a = u * cu
        y = q + jnp.dot(a, w2c[...], preferred_element_type=jnp.float32)
        mu = jnp.mean(y, -1, keepdims=True)
        ycen = y - mu
        r = jax.lax.rsqrt(jnp.mean(ycen * ycen, -1, keepdims=True) + EPS)
        pred = (ycen * r) * gbc[0:1, :] + gbc[1:2, :]
        o_ref[0, 0, i] = pred * g_ref[0, 0, i]

        k = k_ref[0, 0, i]
        v = v_ref[0, 0, i]
        u0 = jnp.dot(k, mw1, preferred_element_type=jnp.float32)
        cu0 = 0.5 * (1.0 + jax.lax.erf(u0 * _INV_SQRT2))
        a0 = u0 * cu0
        y0 = k + jnp.dot(a0, mw2, preferred_element_type=jnp.float32)
        mu0 = jnp.mean(y0, -1, keepdims=True)
        y0c = y0 - mu0
        r0 = jax.lax.rsqrt(jnp.mean(y0c * y0c, -1, keepdims=True) + EPS)
        yh0 = y0c * r0
        pred0 = yh0 * mg_r + mb_r
        e = (pred0 - v) * lr_ref[0, 0, i]
        g_b = jnp.sum(e, 0, keepdims=True)
        g_g = jnp.sum(e * yh0, 0, keepdims=True)
        ge = e * mg_r
        dy = (ge - jnp.mean(ge, -1, keepdims=True)
              - yh0 * jnp.mean(ge * yh0, -1, keepdims=True)) * r0
        gw2 = jax.lax.dot_general(a0, dy, (((0,), (0,)), ((), ())),
                                  preferred_element_type=jnp.float32)
        da = jax.lax.dot_general(dy, mw2, (((1,), (1,)), ((), ())),
                                 preferred_element_type=jnp.float32)
        du = da * (cu0 + u0 * (_INV_SQRT_2PI * jnp.exp(-0.5 * u0 * u0)))
        gw1 = jax.lax.dot_general(k, du, (((0,), (0,)), ((), ())),
                                  preferred_element_type=jnp.float32)

        mom = md_ref[0, p * nc + i]
        dec = md_ref[1, p * nc + i]
        m1[...] = mom * m1[...] - gw1
        m2[...] = mom * m2[...] - gw2
        gbc[2:3, :] = mom * gbc[2:3, :] - g_g
        gbc[3:4, :] = mom * gbc[3:4, :] - g_b
        od = 1.0 - dec
        w1c[...] = od * w1c[...] + m1[...]
        w2c[...] = od * w2c[...] + m2[...]
        gbc[0:1, :] = od * gbc[0:1, :] + gbc[2:3, :]
        gbc[1:2, :] = od * gbc[1:2, :] + gbc[3:4, :]
        return carry

    jax.lax.fori_loop(0, nc, step, 0)


def _comb_body(r_ref, wc_ref, bc_ref, o_ref):
    acc = jnp.dot(r_ref[0, 0], wc_ref[0], preferred_element_type=jnp.float32)
    for h in range(1, HEADS):
        acc = acc + jnp.dot(r_ref[0, h], wc_ref[h], preferred_element_type=jnp.float32)
    o_ref[0] = acc + bc_ref[...]


def kernel(x, g_ret, g_sto, Wq, Wk, Wv, W_lr, b_lr, W_mom, b_mom, W_dec, b_dec,
           W_gate, b_gate, W_comb, b_comb, mw1, mw2, mg, mb):
    b, n, d = x.shape
    nc = n // CHUNK
    f32 = jnp.float32
    x3 = x.reshape(b * nc, CHUNK, d)
    wsm = jnp.concatenate([W_lr, W_gate, W_mom, W_dec], axis=1)
    bsm = jnp.concatenate([b_lr, b_gate, b_mom, b_dec]).reshape(1, 16)
    gsr = g_sto.reshape(1, d)
    grr = g_ret.reshape(1, d)

    k_a, v_a, q_a, lrg, md = pl.pallas_call(
        _prep_body,
        grid=(b * nc,),
        in_specs=[
            pl.BlockSpec((1, CHUNK, d), lambda i: (i, 0, 0)),
            pl.BlockSpec((1, d), lambda i: (0, 0)),
            pl.BlockSpec((1, d), lambda i: (0, 0)),
            pl.BlockSpec((d, HEADS * DH), lambda i: (0, 0)),
            pl.BlockSpec((d, HEADS * DH), lambda i: (0, 0)),
            pl.BlockSpec((d, HEADS * DH), lambda i: (0, 0)),
            pl.BlockSpec((d, 4 * HEADS), lambda i: (0, 0)),
            pl.BlockSpec((1, 4 * HEADS), lambda i: (0, 0)),
        ],
        out_specs=[
            pl.BlockSpec((1, HEADS, 1, CHUNK, DH), lambda i: (i // 32, 0, i % 32, 0, 0)),
            pl.BlockSpec((1, HEADS, 1, CHUNK, DH), lambda i: (i // 32, 0, i % 32, 0, 0)),
            pl.BlockSpec((1, HEADS, 1, CHUNK, DH), lambda i: (i // 32, 0, i % 32, 0, 0)),
            pl.BlockSpec((1, 1, CHUNK, 2 * HEADS), lambda i: (i // 32, i % 32, 0, 0)),
            pl.BlockSpec((1, 1, 1, 2 * HEADS), lambda i: (i // 32, i % 32, 0, 0)),
        ],
        out_shape=[
            jax.ShapeDtypeStruct((b, HEADS, nc, CHUNK, DH), f32),
            jax.ShapeDtypeStruct((b, HEADS, nc, CHUNK, DH), f32),
            jax.ShapeDtypeStruct((b, HEADS, nc, CHUNK, DH), f32),
            jax.ShapeDtypeStruct((b, nc, CHUNK, 2 * HEADS), f32),
            jax.ShapeDtypeStruct((b, nc, 1, 2 * HEADS), f32),
        ],
        compiler_params=pltpu.CompilerParams(
            dimension_semantics=("arbitrary",)),
        name="nm_prep",
    )(x3, gsr, grr, Wq, Wk, Wv, wsm, bsm)

    lrg_t = lrg.transpose(3, 0, 1, 2)[..., None]
    lr_a = lrg_t[:HEADS]
    gate_a = lrg_t[HEADS:]
    md_s = md.reshape(b * nc, 2 * HEADS).transpose(1, 0).reshape(2, HEADS * b * nc)
    mgb = jnp.stack([mg, mb], axis=1)

    r_a = pl.pallas_call(
        _main_body,
        grid=(HEADS * b,),
        in_specs=[
            pl.BlockSpec((1, 1, nc, CHUNK, DH), lambda p: (p % 2, p // 2, 0, 0, 0)),
            pl.BlockSpec((1, 1, nc, CHUNK, DH), lambda p: (p % 2, p // 2, 0, 0, 0)),
            pl.BlockSpec((1, 1, nc, CHUNK, DH), lambda p: (p % 2, p // 2, 0, 0, 0)),
            pl.BlockSpec((1, 1, nc, CHUNK, 1), lambda p: (p // 2, p % 2, 0, 0, 0)),
            pl.BlockSpec((1, 1, nc, CHUNK, 1), lambda p: (p // 2, p % 2, 0, 0, 0)),
            pl.BlockSpec(memory_space=pltpu.SMEM),
            pl.BlockSpec((1, DH, HID), lambda p: (p // 2, 0, 0)),
            pl.BlockSpec((1, HID, DH), lambda p: (p // 2, 0, 0)),
            pl.BlockSpec((1, 2, DH), lambda p: (p // 2, 0, 0)),
        ],
        out_specs=pl.BlockSpec((1, 1, nc, CHUNK, DH), lambda p: (p % 2, p // 2, 0, 0, 0)),
        out_shape=jax.ShapeDtypeStruct((b, HEADS, nc, CHUNK, DH), f32),
        scratch_shapes=[
            pltpu.VMEM((DH, HID), f32),
            pltpu.VMEM((DH, HID), f32),
            pltpu.VMEM((HID, DH), f32),
            pltpu.VMEM((HID, DH), f32),
            pltpu.VMEM((8, DH), f32),
        ],
        compiler_params=pltpu.CompilerParams(
            dimension_semantics=("arbitrary",)),
        name="nm_main",
    )(k_a, v_a, q_a, lr_a, gate_a, md_s, mw1, mw2, mgb)

    r4 = r_a.reshape(b, HEADS, n, DH)
    wc = W_comb.reshape(HEADS, DH, d)
    bc = b_comb.reshape(1, d)
    blkr = 512
    nb = n // blkr
    out = pl.pallas_call(
        _comb_body,
        grid=(b * nb,),
        in_specs=[
            pl.BlockSpec((1, HEADS, blkr, DH), lambda t: (t // nb, 0, t % nb, 0)),
            pl.BlockSpec((HEADS, DH, d), lambda t: (0, 0, 0)),
            pl.BlockSpec((1, d), lambda t: (0, 0)),
        ],
        out_specs=pl.BlockSpec((1, blkr, d), lambda t: (t // nb, t % nb, 0)),
        out_shape=jax.ShapeDtypeStruct((b, n, d), f32),
        compiler_params=pltpu.CompilerParams(
            dimension_semantics=("arbitrary",)),
        name="nm_comb",
    )(r4, wc, bc)
    return out


# speedup vs baseline: 12.1620x; 1.4992x over previous
import jax
import jax.numpy as jnp
from jax.experimental import pallas as pl
from jax.experimental.pallas import tpu as pltpu

DIM = 512
HEADS = 4
DH = 128
HID = 512
CHUNK = 64
MAX_LR = 0.01
EPS = 1e-6
_INV_SQRT2 = 0.7071067811865476
_INV_SQRT_2PI = 0.3989422804014327


def _prep_body(x_ref, gs_ref, gr_ref, wq_ref, wk_ref, wv_ref, wsm_ref, bsm_ref,
               k_ref, v_ref, q_ref, lrg_ref, md_ref):
    xc = x_ref[0]
    rs = jax.lax.rsqrt(jnp.mean(xc * xc, axis=-1, keepdims=True) + EPS)
    xs = xc * rs * gs_ref[...]
    xr = xc * rs * gr_ref[...]
    kall = jnp.dot(xs, wk_ref[...], preferred_element_type=jnp.float32)
    vall = jnp.dot(xs, wv_ref[...], preferred_element_type=jnp.float32)
    qall = jnp.dot(xr, wq_ref[...], preferred_element_type=jnp.float32)
    zs = jnp.dot(xs, wsm_ref[...], preferred_element_type=jnp.float32) + bsm_ref[...]
    zr = jnp.dot(xr, wsm_ref[...], preferred_element_type=jnp.float32) + bsm_ref[...]
    pooled = jnp.mean(xs, axis=0, keepdims=True)
    zp = jnp.dot(pooled, wsm_ref[...], preferred_element_type=jnp.float32) + bsm_ref[...]
    for h in range(HEADS):
        sl = slice(h * DH, (h + 1) * DH)
        kh = kall[:, sl]
        k_ref[0, h, 0] = kh * jax.lax.rsqrt(jnp.sum(kh * kh, -1, keepdims=True) + 1e-12)
        qh = qall[:, sl]
        q_ref[0, h, 0] = qh * jax.lax.rsqrt(jnp.sum(qh * qh, -1, keepdims=True) + 1e-12)
        v_ref[0, h, 0] = vall[:, sl]
    lr = jax.nn.sigmoid(zs[:, 0:HEADS]) * (2.0 * MAX_LR / DH)
    gate = jax.nn.sigmoid(zr[:, HEADS:2 * HEADS])
    lrg_ref[0, 0] = jnp.concatenate([lr, gate], axis=-1)
    md_ref[0, 0] = jax.nn.sigmoid(zp[:, 2 * HEADS:4 * HEADS])


def _main_body(k_ref, v_ref, q_ref, lr_ref, g_ref, md_ref, mw1_ref, mw2_ref,
               mgb_ref, o_ref, w1c, m1, w2c, m2, gbc, a0s, dus, dys, ggs, gbs):
    p = pl.program_id(0)
    nb = k_ref.shape[0]
    nc = k_ref.shape[2]
    w1c[...] = jnp.broadcast_to(mw1_ref[0][None], (nb, DH, HID))
    m1[...] = jnp.zeros_like(m1)
    w2c[...] = jnp.broadcast_to(mw2_ref[0][None], (nb, HID, DH))
    m2[...] = jnp.zeros_like(m2)
    for g in range(nb):
        gbc[g, 0:2, :] = mgb_ref[0]
        gbc[g, 2:4, :] = jnp.zeros((2, DH), jnp.float32)
    mw1 = mw1_ref[0]
    mw2 = mw2_ref[0]
    mg_r = mgb_ref[0, 0:1, :]
    mb_r = mgb_ref[0, 1:2, :]

    ntok = nb * nc * CHUNK
    kf = k_ref[...].reshape(ntok, DH)
    vf = v_ref[...].reshape(ntok, DH)
    lrf = lr_ref[...].reshape(ntok, 1)
    u0 = jnp.dot(kf, mw1, preferred_element_type=jnp.float32)
    cu0 = 0.5 * (1.0 + jax.lax.erf(u0 * _INV_SQRT2))
    a0 = u0 * cu0
    y0 = kf + jnp.dot(a0, mw2, preferred_element_type=jnp.float32)
    mu0 = jnp.mean(y0, -1, keepdims=True)
    y0c = y0 - mu0
    r0 = jax.lax.rsqrt(jnp.mean(y0c * y0c, -1, keepdims=True) + EPS)
    yh0 = y0c * r0
    e = (yh0 * mg_r + mb_r - vf) * lrf
    eyh = e * yh0
    ge = e * mg_r
    dy = (ge - jnp.mean(ge, -1, keepdims=True)
          - yh0 * jnp.mean(ge * yh0, -1, keepdims=True)) * r0
    da = jax.lax.dot_general(dy, mw2, (((1,), (1,)), ((), ())),
                             preferred_element_type=jnp.float32)
    du = da * (cu0 + u0 * (_INV_SQRT_2PI * jnp.exp(-0.5 * u0 * u0)))
    a0s[...] = a0.reshape(nb, nc, CHUNK, HID).astype(jnp.bfloat16)
    dus[...] = du.reshape(nb, nc, CHUNK, HID).astype(jnp.bfloat16)
    dys[...] = dy.reshape(nb, nc, CHUNK, DH).astype(jnp.bfloat16)
    ggs[...] = jnp.sum(eyh.reshape(nb, nc, CHUNK, DH), axis=2, keepdims=True)
    gbs[...] = jnp.sum(e.reshape(nb, nc, CHUNK, DH), axis=2, keepdims=True)

    def step(i, carry):
        for g in range(nb):
            q = q_ref[g, 0, i]
            u = jnp.dot(q, w1c[g], preferred_element_type=jnp.float32)
            cu = 0.5 * (1.0 + jax.lax.erf(u * _INV_SQRT2))
            a = u * cu
            y = q + jnp.dot(a, w2c[g], preferred_element_type=jnp.float32)
            mu = jnp.mean(y, -1, keepdims=True)
            ycen = y - mu
            r = jax.lax.rsqrt(jnp.mean(ycen * ycen, -1, keepdims=True) + EPS)
            pred = (ycen * r) * gbc[g, 0:1, :] + gbc[g, 1:2, :]
            o_ref[g, 0, i] = pred * g_ref[0, g, i]

            kb = k_ref[g, 0, i].astype(jnp.bfloat16)
            gw2 = jax.lax.dot_general(a0s[g, i], dys[g, i], (((0,), (0,)), ((), ())),
                                      preferred_element_type=jnp.float32)
            gw1 = jax.lax.dot_general(kb, dus[g, i], (((0,), (0,)), ((), ())),
                                      preferred_element_type=jnp.float32)
            g_g = ggs[g, i]
            g_b = gbs[g, i]

            mom = md_ref[0, (p * nb + g) * nc + i]
            dec = md_ref[1, (p * nb + g) * nc + i]
            m1[g] = mom * m1[g] - gw1
            m2[g] = mom * m2[g] - gw2
            gbc[g, 2:3, :] = mom * gbc[g, 2:3, :] - g_g
            gbc[g, 3:4, :] = mom * gbc[g, 3:4, :] - g_b
            od = 1.0 - dec
            w1c[g] = od * w1c[g] + m1[g]
            w2c[g] = od * w2c[g] + m2[g]
            gbc[g, 0:1, :] = od * gbc[g, 0:1, :] + gbc[g, 2:3, :]
            gbc[g, 1:2, :] = od * gbc[g, 1:2, :] + gbc[g, 3:4, :]
        return carry

    jax.lax.fori_loop(0, nc, step, 0)


def _comb_body(r_ref, wc_ref, bc_ref, o_ref):
    acc = jnp.dot(r_ref[0, 0], wc_ref[0], preferred_element_type=jnp.float32)
    for h in range(1, HEADS):
        acc = acc + jnp.dot(r_ref[0, h], wc_ref[h], preferred_element_type=jnp.float32)
    o_ref[0] = acc + bc_ref[...]


def kernel(x, g_ret, g_sto, Wq, Wk, Wv, W_lr, b_lr, W_mom, b_mom, W_dec, b_dec,
           W_gate, b_gate, W_comb, b_comb, mw1, mw2, mg, mb):
    b, n, d = x.shape
    nc = n // CHUNK
    f32 = jnp.float32
    x3 = x.reshape(b * nc, CHUNK, d)
    wsm = jnp.concatenate([W_lr, W_gate, W_mom, W_dec], axis=1)
    bsm = jnp.concatenate([b_lr, b_gate, b_mom, b_dec]).reshape(1, 16)
    gsr = g_sto.reshape(1, d)
    grr = g_ret.reshape(1, d)

    k_a, v_a, q_a, lrg, md = pl.pallas_call(
        _prep_body,
        grid=(b * nc,),
        in_specs=[
            pl.BlockSpec((1, CHUNK, d), lambda i: (i, 0, 0)),
            pl.BlockSpec((1, d), lambda i: (0, 0)),
            pl.BlockSpec((1, d), lambda i: (0, 0)),
            pl.BlockSpec((d, HEADS * DH), lambda i: (0, 0)),
            pl.BlockSpec((d, HEADS * DH), lambda i: (0, 0)),
            pl.BlockSpec((d, HEADS * DH), lambda i: (0, 0)),
            pl.BlockSpec((d, 4 * HEADS), lambda i: (0, 0)),
            pl.BlockSpec((1, 4 * HEADS), lambda i: (0, 0)),
        ],
        out_specs=[
            pl.BlockSpec((1, HEADS, 1, CHUNK, DH), lambda i: (i // 32, 0, i % 32, 0, 0)),
            pl.BlockSpec((1, HEADS, 1, CHUNK, DH), lambda i: (i // 32, 0, i % 32, 0, 0)),
            pl.BlockSpec((1, HEADS, 1, CHUNK, DH), lambda i: (i // 32, 0, i % 32, 0, 0)),
            pl.BlockSpec((1, 1, CHUNK, 2 * HEADS), lambda i: (i // 32, i % 32, 0, 0)),
            pl.BlockSpec((1, 1, 1, 2 * HEADS), lambda i: (i // 32, i % 32, 0, 0)),
        ],
        out_shape=[
            jax.ShapeDtypeStruct((b, HEADS, nc, CHUNK, DH), f32),
            jax.ShapeDtypeStruct((b, HEADS, nc, CHUNK, DH), f32),
            jax.ShapeDtypeStruct((b, HEADS, nc, CHUNK, DH), f32),
            jax.ShapeDtypeStruct((b, nc, CHUNK, 2 * HEADS), f32),
            jax.ShapeDtypeStruct((b, nc, 1, 2 * HEADS), f32),
        ],
        compiler_params=pltpu.CompilerParams(
            dimension_semantics=("arbitrary",)),
        name="nm_prep",
    )(x3, gsr, grr, Wq, Wk, Wv, wsm, bsm)

    lrg_t = lrg.transpose(3, 0, 1, 2)[..., None]
    lr_a = lrg_t[:HEADS]
    gate_a = lrg_t[HEADS:]
    md_s = md.reshape(b * nc, 2 * HEADS).transpose(1, 0).reshape(2, HEADS * b * nc)
    mgb = jnp.stack([mg, mb], axis=1)

    r_a = pl.pallas_call(
        _main_body,
        grid=(HEADS,),
        in_specs=[
            pl.BlockSpec((b, 1, nc, CHUNK, DH), lambda p: (0, p, 0, 0, 0)),
            pl.BlockSpec((b, 1, nc, CHUNK, DH), lambda p: (0, p, 0, 0, 0)),
            pl.BlockSpec((b, 1, nc, CHUNK, DH), lambda p: (0, p, 0, 0, 0)),
            pl.BlockSpec((1, b, nc, CHUNK, 1), lambda p: (p, 0, 0, 0, 0)),
            pl.BlockSpec((1, b, nc, CHUNK, 1), lambda p: (p, 0, 0, 0, 0)),
            pl.BlockSpec(memory_space=pltpu.SMEM),
            pl.BlockSpec((1, DH, HID), lambda p: (p, 0, 0)),
            pl.BlockSpec((1, HID, DH), lambda p: (p, 0, 0)),
            pl.BlockSpec((1, 2, DH), lambda p: (p, 0, 0)),
        ],
        out_specs=pl.BlockSpec((b, 1, nc, CHUNK, DH), lambda p: (0, p, 0, 0, 0)),
        out_shape=jax.ShapeDtypeStruct((b, HEADS, nc, CHUNK, DH), f32),
        scratch_shapes=[
            pltpu.VMEM((b, DH, HID), f32),
            pltpu.VMEM((b, DH, HID), f32),
            pltpu.VMEM((b, HID, DH), f32),
            pltpu.VMEM((b, HID, DH), f32),
            pltpu.VMEM((b, 8, DH), f32),
            pltpu.VMEM((b, nc, CHUNK, HID), jnp.bfloat16),
            pltpu.VMEM((b, nc, CHUNK, HID), jnp.bfloat16),
            pltpu.VMEM((b, nc, CHUNK, DH), jnp.bfloat16),
            pltpu.VMEM((b, nc, 1, DH), f32),
            pltpu.VMEM((b, nc, 1, DH), f32),
        ],
        compiler_params=pltpu.CompilerParams(
            dimension_semantics=("arbitrary",),
            vmem_limit_bytes=100 * 1024 * 1024),
        name="nm_main",
    )(k_a, v_a, q_a, lr_a, gate_a, md_s, mw1, mw2, mgb)

    r4 = r_a.reshape(b, HEADS, n, DH)
    wc = W_comb.reshape(HEADS, DH, d)
    bc = b_comb.reshape(1, d)
    blkr = 512
    nb = n // blkr
    out = pl.pallas_call(
        _comb_body,
        grid=(b * nb,),
        in_specs=[
            pl.BlockSpec((1, HEADS, blkr, DH), lambda t: (t // nb, 0, t % nb, 0)),
            pl.BlockSpec((HEADS, DH, d), lambda t: (0, 0, 0)),
            pl.BlockSpec((1, d), lambda t: (0, 0)),
        ],
        out_specs=pl.BlockSpec((1, blkr, d), lambda t: (t // nb, t % nb, 0)),
        out_shape=jax.ShapeDtypeStruct((b, n, d), f32),
        compiler_params=pltpu.CompilerParams(
            dimension_semantics=("arbitrary",)),
        name="nm_comb",
    )(r4, wc, bc)
    return out


# speedup vs baseline: 15.4341x; 1.2690x over previous
import jax
import jax.numpy as jnp
from jax.experimental import pallas as pl
from jax.experimental.pallas import tpu as pltpu

DIM = 512
HEADS = 4
DH = 128
HID = 512
CHUNK = 64
MAX_LR = 0.01
EPS = 1e-6
_INV_SQRT2 = 0.7071067811865476
_INV_SQRT_2PI = 0.3989422804014327


_CB = 4


def _prep_body(x_ref, gs_ref, gr_ref, wq_ref, wk_ref, wv_ref, wsm_ref, bsm_ref,
               k_ref, v_ref, q_ref, lrg_ref, md_ref):
    xc = x_ref[0]
    nt = _CB * CHUNK
    rs = jax.lax.rsqrt(jnp.mean(xc * xc, axis=-1, keepdims=True) + EPS)
    xs = xc * rs * gs_ref[...]
    xr = xc * rs * gr_ref[...]
    kall = jnp.dot(xs, wk_ref[...], preferred_element_type=jnp.float32)
    vall = jnp.dot(xs, wv_ref[...], preferred_element_type=jnp.float32)
    qall = jnp.dot(xr, wq_ref[...], preferred_element_type=jnp.float32)
    zs = jnp.dot(xs, wsm_ref[...], preferred_element_type=jnp.float32) + bsm_ref[...]
    zr = jnp.dot(xr, wsm_ref[...], preferred_element_type=jnp.float32) + bsm_ref[...]
    pooled = jnp.mean(xs.reshape(_CB, CHUNK, DIM), axis=1)
    zp = jnp.dot(pooled, wsm_ref[...], preferred_element_type=jnp.float32) + bsm_ref[...]
    for h in range(HEADS):
        sl = slice(h * DH, (h + 1) * DH)
        kh = kall[:, sl]
        kh = kh * jax.lax.rsqrt(jnp.sum(kh * kh, -1, keepdims=True) + 1e-12)
        k_ref[0, h] = kh.reshape(_CB, CHUNK, DH)
        qh = qall[:, sl]
        qh = qh * jax.lax.rsqrt(jnp.sum(qh * qh, -1, keepdims=True) + 1e-12)
        q_ref[0, h] = qh.reshape(_CB, CHUNK, DH)
        v_ref[0, h] = vall[:, sl].reshape(_CB, CHUNK, DH)
    lr = jax.nn.sigmoid(zs[:, 0:HEADS]) * (2.0 * MAX_LR / DH)
    gate = jax.nn.sigmoid(zr[:, HEADS:2 * HEADS])
    lrg_ref[0] = jnp.concatenate([lr, gate], axis=-1).reshape(_CB, CHUNK, 2 * HEADS)
    md_ref[0] = jax.nn.sigmoid(zp[:, 2 * HEADS:4 * HEADS]).reshape(_CB, 1, 2 * HEADS)


def _main_body(k_ref, v_ref, q_ref, lr_ref, g_ref, md_ref, mw1_ref, mw2_ref,
               mgb_ref, o_ref, w1c, m1, w2c, m2, gbc, a0s, dus, dys, ggs, gbs):
    p = pl.program_id(0)
    nb = k_ref.shape[0]
    nc = k_ref.shape[2]
    w1c[...] = jnp.broadcast_to(mw1_ref[0][None], (nb, DH, HID))
    m1[...] = jnp.zeros_like(m1)
    w2c[...] = jnp.broadcast_to(mw2_ref[0][None], (nb, HID, DH))
    m2[...] = jnp.zeros_like(m2)
    for g in range(nb):
        gbc[g, 0:2, :] = mgb_ref[0]
        gbc[g, 2:4, :] = jnp.zeros((2, DH), jnp.float32)
    mw1 = mw1_ref[0]
    mw2 = mw2_ref[0]
    mg_r = mgb_ref[0, 0:1, :]
    mb_r = mgb_ref[0, 1:2, :]

    ntok = nb * nc * CHUNK
    kf = k_ref[...].reshape(ntok, DH)
    vf = v_ref[...].reshape(ntok, DH)
    lrf = lr_ref[...].reshape(ntok, 1)
    u0 = jnp.dot(kf, mw1, preferred_element_type=jnp.float32)
    cu0 = 0.5 * (1.0 + jax.lax.erf(u0 * _INV_SQRT2))
    a0 = u0 * cu0
    y0 = kf + jnp.dot(a0, mw2, preferred_element_type=jnp.float32)
    mu0 = jnp.mean(y0, -1, keepdims=True)
    y0c = y0 - mu0
    r0 = jax.lax.rsqrt(jnp.mean(y0c * y0c, -1, keepdims=True) + EPS)
    yh0 = y0c * r0
    e = (yh0 * mg_r + mb_r - vf) * lrf
    eyh = e * yh0
    ge = e * mg_r
    dy = (ge - jnp.mean(ge, -1, keepdims=True)
          - yh0 * jnp.mean(ge * yh0, -1, keepdims=True)) * r0
    da = jax.lax.dot_general(dy, mw2, (((1,), (1,)), ((), ())),
                             preferred_element_type=jnp.float32)
    du = da * (cu0 + u0 * (_INV_SQRT_2PI * jnp.exp(-0.5 * u0 * u0)))
    a0s[...] = a0.reshape(nb, nc, CHUNK, HID).astype(jnp.bfloat16)
    dus[...] = du.reshape(nb, nc, CHUNK, HID).astype(jnp.bfloat16)
    dys[...] = dy.reshape(nb, nc, CHUNK, DH).astype(jnp.bfloat16)
    ggs[...] = jnp.sum(eyh.reshape(nb, nc, CHUNK, DH), axis=2, keepdims=True)
    gbs[...] = jnp.sum(e.reshape(nb, nc, CHUNK, DH), axis=2, keepdims=True)

    def step(i, carry):
        for g in range(nb):
            q = q_ref[g, 0, i]
            u = jnp.dot(q, w1c[g], preferred_element_type=jnp.float32)
            cu = 0.5 * (1.0 + jax.lax.erf(u * _INV_SQRT2))
            a = u * cu
            y = q + jnp.dot(a, w2c[g], preferred_element_type=jnp.float32)
            mu = jnp.mean(y, -1, keepdims=True)
            ycen = y - mu
            r = jax.lax.rsqrt(jnp.mean(ycen * ycen, -1, keepdims=True) + EPS)
            pred = (ycen * r) * gbc[g, 0:1, :] + gbc[g, 1:2, :]
            o_ref[g, 0, i] = pred * g_ref[0, g, i]

            kb = k_ref[g, 0, i].astype(jnp.bfloat16)
            gw2 = jax.lax.dot_general(a0s[g, i], dys[g, i], (((0,), (0,)), ((), ())),
                                      preferred_element_type=jnp.float32)
            gw1 = jax.lax.dot_general(kb, dus[g, i], (((0,), (0,)), ((), ())),
                                      preferred_element_type=jnp.float32)
            g_g = ggs[g, i]
            g_b = gbs[g, i]

            mom = md_ref[0, (p * nb + g) * nc + i]
            dec = md_ref[1, (p * nb + g) * nc + i]
            m1[g] = mom * m1[g] - gw1
            m2[g] = mom * m2[g] - gw2
            gbc[g, 2:3, :] = mom * gbc[g, 2:3, :] - g_g
            gbc[g, 3:4, :] = mom * gbc[g, 3:4, :] - g_b
            od = 1.0 - dec
            w1c[g] = od * w1c[g] + m1[g]
            w2c[g] = od * w2c[g] + m2[g]
            gbc[g, 0:1, :] = od * gbc[g, 0:1, :] + gbc[g, 2:3, :]
            gbc[g, 1:2, :] = od * gbc[g, 1:2, :] + gbc[g, 3:4, :]
        return carry

    jax.lax.fori_loop(0, nc, step, 0)


def _comb_body(r_ref, wc_ref, bc_ref, o_ref):
    acc = jnp.dot(r_ref[0, 0], wc_ref[0], preferred_element_type=jnp.float32)
    for h in range(1, HEADS):
        acc = acc + jnp.dot(r_ref[0, h], wc_ref[h], preferred_element_type=jnp.float32)
    o_ref[0] = acc + bc_ref[...]


def kernel(x, g_ret, g_sto, Wq, Wk, Wv, W_lr, b_lr, W_mom, b_mom, W_dec, b_dec,
           W_gate, b_gate, W_comb, b_comb, mw1, mw2, mg, mb):
    b, n, d = x.shape
    nc = n // CHUNK
    f32 = jnp.float32
    x3 = x.reshape(b * (nc // _CB), _CB * CHUNK, d)
    wsm = jnp.concatenate([W_lr, W_gate, W_mom, W_dec], axis=1)
    bsm = jnp.concatenate([b_lr, b_gate, b_mom, b_dec]).reshape(1, 16)
    gsr = g_sto.reshape(1, d)
    grr = g_ret.reshape(1, d)

    ncb = nc // _CB
    k_a, v_a, q_a, lrg, md = pl.pallas_call(
        _prep_body,
        grid=(b * ncb,),
        in_specs=[
            pl.BlockSpec((1, _CB * CHUNK, d), lambda i: (i, 0, 0)),
            pl.BlockSpec((1, d), lambda i: (0, 0)),
            pl.BlockSpec((1, d), lambda i: (0, 0)),
            pl.BlockSpec((d, HEADS * DH), lambda i: (0, 0)),
            pl.BlockSpec((d, HEADS * DH), lambda i: (0, 0)),
            pl.BlockSpec((d, HEADS * DH), lambda i: (0, 0)),
            pl.BlockSpec((d, 4 * HEADS), lambda i: (0, 0)),
            pl.BlockSpec((1, 4 * HEADS), lambda i: (0, 0)),
        ],
        out_specs=[
            pl.BlockSpec((1, HEADS, _CB, CHUNK, DH), lambda i: (i // 8, 0, i % 8, 0, 0)),
            pl.BlockSpec((1, HEADS, _CB, CHUNK, DH), lambda i: (i // 8, 0, i % 8, 0, 0)),
            pl.BlockSpec((1, HEADS, _CB, CHUNK, DH), lambda i: (i // 8, 0, i % 8, 0, 0)),
            pl.BlockSpec((1, _CB, CHUNK, 2 * HEADS), lambda i: (i // 8, i % 8, 0, 0)),
            pl.BlockSpec((1, _CB, 1, 2 * HEADS), lambda i: (i // 8, i % 8, 0, 0)),
        ],
        out_shape=[
            jax.ShapeDtypeStruct((b, HEADS, nc, CHUNK, DH), f32),
            jax.ShapeDtypeStruct((b, HEADS, nc, CHUNK, DH), f32),
            jax.ShapeDtypeStruct((b, HEADS, nc, CHUNK, DH), f32),
            jax.ShapeDtypeStruct((b, nc, CHUNK, 2 * HEADS), f32),
            jax.ShapeDtypeStruct((b, nc, 1, 2 * HEADS), f32),
        ],
        compiler_params=pltpu.CompilerParams(
            dimension_semantics=("arbitrary",)),
        name="nm_prep",
    )(x3, gsr, grr, Wq, Wk, Wv, wsm, bsm)

    lrg_t = lrg.transpose(3, 0, 1, 2)[..., None]
    lr_a = lrg_t[:HEADS]
    gate_a = lrg_t[HEADS:]
    md_s = md.reshape(b * nc, 2 * HEADS).transpose(1, 0).reshape(2, HEADS * b * nc)
    mgb = jnp.stack([mg, mb], axis=1)

    r_a = pl.pallas_call(
        _main_body,
        grid=(HEADS,),
        in_specs=[
            pl.BlockSpec((b, 1, nc, CHUNK, DH), lambda p: (0, p, 0, 0, 0)),
            pl.BlockSpec((b, 1, nc, CHUNK, DH), lambda p: (0, p, 0, 0, 0)),
            pl.BlockSpec((b, 1, nc, CHUNK, DH), lambda p: (0, p, 0, 0, 0)),
            pl.BlockSpec((1, b, nc, CHUNK, 1), lambda p: (p, 0, 0, 0, 0)),
            pl.BlockSpec((1, b, nc, CHUNK, 1), lambda p: (p, 0, 0, 0, 0)),
            pl.BlockSpec(memory_space=pltpu.SMEM),
            pl.BlockSpec((1, DH, HID), lambda p: (p, 0, 0)),
            pl.BlockSpec((1, HID, DH), lambda p: (p, 0, 0)),
            pl.BlockSpec((1, 2, DH), lambda p: (p, 0, 0)),
        ],
        out_specs=pl.BlockSpec((b, 1, nc, CHUNK, DH), lambda p: (0, p, 0, 0, 0)),
        out_shape=jax.ShapeDtypeStruct((b, HEADS, nc, CHUNK, DH), f32),
        scratch_shapes=[
            pltpu.VMEM((b, DH, HID), f32),
            pltpu.VMEM((b, DH, HID), f32),
            pltpu.VMEM((b, HID, DH), f32),
            pltpu.VMEM((b, HID, DH), f32),
            pltpu.VMEM((b, 8, DH), f32),
            pltpu.VMEM((b, nc, CHUNK, HID), jnp.bfloat16),
            pltpu.VMEM((b, nc, CHUNK, HID), jnp.bfloat16),
            pltpu.VMEM((b, nc, CHUNK, DH), jnp.bfloat16),
            pltpu.VMEM((b, nc, 1, DH), f32),
            pltpu.VMEM((b, nc, 1, DH), f32),
        ],
        compiler_params=pltpu.CompilerParams(
            dimension_semantics=("arbitrary",),
            vmem_limit_bytes=100 * 1024 * 1024),
        name="nm_main",
    )(k_a, v_a, q_a, lr_a, gate_a, md_s, mw1, mw2, mgb)

    r4 = r_a.reshape(b, HEADS, n, DH)
    wc = W_comb.reshape(HEADS, DH, d)
    bc = b_comb.reshape(1, d)
    blkr = 512
    nb = n // blkr
    out = pl.pallas_call(
        _comb_body,
        grid=(b * nb,),
        in_specs=[
            pl.BlockSpec((1, HEADS, blkr, DH), lambda t: (t // nb, 0, t % nb, 0)),
            pl.BlockSpec((HEADS, DH, d), lambda t: (0, 0, 0)),
            pl.BlockSpec((1, d), lambda t: (0, 0)),
        ],
        out_specs=pl.BlockSpec((1, blkr, d), lambda t: (t // nb, t % nb, 0)),
        out_shape=jax.ShapeDtypeStruct((b, n, d), f32),
        compiler_params=pltpu.CompilerParams(
            dimension_semantics=("arbitrary",)),
        name="nm_comb",
    )(r4, wc, bc)
    return out
